# baseline (speedup 1.0000x reference)
import functools

import jax
import jax.numpy as jnp
from jax import lax
from jax.experimental import pallas as pl
from jax.experimental.pallas import tpu as pltpu
from jax.experimental.pallas import tpu_sc as plsc

N = 10000
E = 320000
D = 128
H = 128
C = 7

NC = 2
NS = 16
NW = NC * NS

CHUNK = 128
NCH = E // CHUNK
CPW = 80
PADC = NW * CPW - NCH
NP = 10240
RPT = NP // NS
IB = 40
DEG_W = 128


def _zero_fill(ref, nrows, ncols):
    def row(i, _):
        for k in range(ncols // 16):
            ref[i, pl.ds(k * 16, 16)] = jnp.zeros((16,), jnp.float32)
        return 0
    lax.fori_loop(0, nrows, row, 0)


def _deg_body(ei_hbm, out_hbm, idx_v, ones_v, zero_v, acc_sh):
    c = lax.axis_index("c")
    s = lax.axis_index("s")
    wid = c * NS + s

    def ones_row(i, _):
        for k in range(DEG_W // 16):
            ones_v[i, pl.ds(k * 16, 16)] = jnp.full((16,), 1.0, jnp.float32)
        return 0
    lax.fori_loop(0, CHUNK, ones_row, 0)
    _zero_fill(zero_v, 64, DEG_W)

    def zero_acc(k, _):
        pltpu.sync_copy(zero_v, acc_sh.at[pl.ds(s * RPT + k * 64, 64)])
        return 0
    lax.fori_loop(0, RPT // 64, zero_acc, 0)
    plsc.subcore_barrier()

    pltpu.sync_copy(ei_hbm.at[1, pl.ds(wid * CPW, CPW), :], idx_v)

    def body(j, _):
        pltpu.sync_copy(ones_v, acc_sh.at[idx_v.at[j]], add=True)
        return 0
    lax.fori_loop(0, CPW, body, 0)
    plsc.subcore_barrier()

    pltpu.sync_copy(acc_sh.at[pl.ds(s * RPT, RPT)],
                    out_hbm.at[c, pl.ds(s * RPT, RPT)])


def _agg_body(hs_hbm, src_hbm, dst_hbm, out_hbm,
              src_v, dst_v, rows0, rows1, acc_sh,
              gsem0, gsem1, ssem0, ssem1):
    c = lax.axis_index("c")
    s = lax.axis_index("s")
    wid = c * NS + s

    _zero_fill(rows0, CHUNK, H)

    def zero_acc(k, _):
        pltpu.sync_copy(rows0, acc_sh.at[pl.ds(s * RPT + k * CHUNK, CHUNK)])
        return 0
    lax.fori_loop(0, RPT // CHUNK, zero_acc, 0)
    plsc.subcore_barrier()

    rows = (rows0, rows1)
    gsems = (gsem0, gsem1)
    ssems = (ssem0, ssem1)

    def sidx(r):
        return src_v.at[r]

    def didx(r):
        return dst_v.at[r]

    for q in range(CPW // IB):
        pltpu.sync_copy(src_hbm.at[pl.ds(wid * CPW + q * IB, IB)], src_v)
        pltpu.sync_copy(dst_hbm.at[pl.ds(wid * CPW + q * IB, IB)], dst_v)
        for b in range(2):
            pltpu.async_copy(hs_hbm.at[sidx(b)], rows[b], gsems[b])

        def pair(p, _):
            for b in range(2):
                r = 2 * p + b
                pltpu.make_async_copy(hs_hbm.at[sidx(0)],
                                      rows[b], gsems[b]).wait()
                pltpu.async_copy(rows[b], acc_sh.at[didx(r)], ssems[b],
                                 add=True)
                nr = r + 2

                @pl.when(nr < IB)
                def _():
                    pltpu.make_async_copy(rows[b], acc_sh.at[didx(r)],
                                          ssems[b]).wait()
                    pltpu.async_copy(hs_hbm.at[sidx(nr)], rows[b], gsems[b])
            return 0
        lax.fori_loop(0, IB // 2, pair, 0)
        for b in range(2):
            pltpu.make_async_copy(rows[b], acc_sh.at[didx(0)],
                                  ssems[b]).wait()
    plsc.subcore_barrier()

    pltpu.sync_copy(acc_sh.at[pl.ds(s * RPT, RPT)],
                    out_hbm.at[c, pl.ds(s * RPT, RPT)])


@functools.cache
def _sc_kernels():
    mesh = plsc.VectorSubcoreMesh(core_axis_name="c", subcore_axis_name="s",
                                  num_cores=NC, num_subcores=NS)
    deg_k = pl.kernel(
        _deg_body,
        out_type=jax.ShapeDtypeStruct((NC, NP, DEG_W), jnp.float32),
        mesh=mesh,
        scratch_types=[
            pltpu.VMEM((CPW, CHUNK), jnp.int32),
            pltpu.VMEM((CHUNK, DEG_W), jnp.float32),
            pltpu.VMEM((64, DEG_W), jnp.float32),
            pltpu.VMEM_SHARED((NP, DEG_W), jnp.float32),
        ],
    )
    agg_k = pl.kernel(
        _agg_body,
        out_type=jax.ShapeDtypeStruct((NC, NP, H), jnp.float32),
        mesh=mesh,
        scratch_types=[
            pltpu.VMEM((IB, CHUNK), jnp.int32),
            pltpu.VMEM((IB, CHUNK), jnp.int32),
            pltpu.VMEM((CHUNK, H), jnp.float32),
            pltpu.VMEM((CHUNK, H), jnp.float32),
            pltpu.VMEM_SHARED((NP, H), jnp.float32),
            pltpu.SemaphoreType.DMA,
            pltpu.SemaphoreType.DMA,
            pltpu.SemaphoreType.DMA,
            pltpu.SemaphoreType.DMA,
        ],
    )
    return deg_k, agg_k


BLK = 2048


def _k1_body(degtab_ref, x_ref, w1_ref, hs_ref, dinv_ref):
    deg = degtab_ref[0, :, 0:1] + degtab_ref[1, :, 0:1]
    dinv = lax.rsqrt(deg + 1.0)
    h = jnp.dot(x_ref[...], w1_ref[...], preferred_element_type=jnp.float32)
    hs_ref[...] = h * dinv
    dinv_ref[...] = dinv


def _k2_body(acc_ref, hs_ref, dinv_ref, b_ref, w_ref, out_ref):
    dinv = dinv_ref[...]
    z = jnp.maximum((acc_ref[0] + acc_ref[1] + hs_ref[...]) * dinv + b_ref[...],
                    0.0)
    h2 = jnp.dot(z, w_ref[...], preferred_element_type=jnp.float32)
    out_ref[...] = h2 * dinv


def _k3_body(acc_ref, hs_ref, dinv_ref, b_ref, w3_ref, b3_ref, out_ref):
    dinv = dinv_ref[...]
    z = jnp.maximum((acc_ref[0] + acc_ref[1] + hs_ref[...]) * dinv + b_ref[...],
                    0.0)
    logits = jnp.dot(z, w3_ref[...], preferred_element_type=jnp.float32)
    logits = logits + b3_ref[...]
    m = jnp.max(logits, axis=1, keepdims=True)
    e = jnp.exp(logits - m)
    sm = e / jnp.sum(e, axis=1, keepdims=True)
    out_ref[...] = sm[:, :8]


_row_spec = pl.BlockSpec((BLK, 128), lambda i: (i, 0))
_vec_spec = pl.BlockSpec((BLK, 1), lambda i: (i, 0))
_mat_spec = pl.BlockSpec((128, 128), lambda i: (0, 0))
_bias_spec = pl.BlockSpec((1, 128), lambda i: (0, 0))
_acc_spec = pl.BlockSpec((NC, BLK, 128), lambda i: (0, i, 0))

_k1 = pl.pallas_call(
    _k1_body,
    grid=(NP // BLK,),
    in_specs=[_acc_spec, _row_spec, _mat_spec],
    out_specs=[_row_spec, _vec_spec],
    out_shape=[jax.ShapeDtypeStruct((NP, H), jnp.float32),
               jax.ShapeDtypeStruct((NP, 1), jnp.float32)],
)

_k2 = pl.pallas_call(
    _k2_body,
    grid=(NP // BLK,),
    in_specs=[_acc_spec, _row_spec, _vec_spec, _bias_spec, _mat_spec],
    out_specs=_row_spec,
    out_shape=jax.ShapeDtypeStruct((NP, H), jnp.float32),
)

_k3 = pl.pallas_call(
    _k3_body,
    grid=(NP // BLK,),
    in_specs=[_acc_spec, _row_spec, _vec_spec, _bias_spec, _mat_spec,
              _bias_spec],
    out_specs=pl.BlockSpec((BLK, 8), lambda i: (i, 0)),
    out_shape=jax.ShapeDtypeStruct((NP, 8), jnp.float32),
)


def kernel(x, edge_index, W1, b1, W2, b2, W3, b3):
    ei3 = edge_index.reshape(2, NCH, CHUNK)
    pad_dst = N + jnp.arange(PADC * CHUNK, dtype=jnp.int32) % (NP - N)
    pad3 = jnp.stack([jnp.zeros((PADC, CHUNK), jnp.int32),
                      pad_dst.reshape(PADC, CHUNK)])
    ei4 = jnp.concatenate([ei3, pad3], axis=1)
    x_p = jnp.pad(x, ((0, NP - N), (0, 0)))

    src4 = ei4[0]
    dst4 = ei4[1]

    _deg_kernel, _agg_kernel = _sc_kernels()
    degtab = _deg_kernel(ei4)

    hs1, dinv = _k1(degtab, x_p, W1)
    acc1 = _agg_kernel(hs1, src4, dst4)
    hs2 = _k2(acc1, hs1, dinv, b1.reshape(1, H), W2)
    acc2 = _agg_kernel(hs2, src4, dst4)

    w3p = jnp.pad(W3, ((0, 0), (0, 128 - C)))
    b3p = jnp.concatenate([b3, jnp.full((128 - C,), -1e30, jnp.float32)])
    out = _k3(acc2, hs2, dinv, b2.reshape(1, H), w3p, b3p.reshape(1, 128))
    return out[:N, :C]


# speedup vs baseline: 2.9116x; 2.9116x over previous
import functools

import jax
import jax.numpy as jnp
from jax import lax
from jax.experimental import pallas as pl
from jax.experimental.pallas import tpu as pltpu
from jax.experimental.pallas import tpu_sc as plsc

N = 10000
E = 320000
D = 128
H = 128
C = 7

NC = 2
NS = 16
NW = NC * NS

CHUNK = 128
NCH = E // CHUNK
CPW = 80
PADC = NW * CPW - NCH
NP = 10240
RPT = NP // NS
IB = 40
DEG_W = 128


def _zero_fill(ref, nrows, ncols):
    def row(i, _):
        for k in range(ncols // 16):
            ref[i, pl.ds(k * 16, 16)] = jnp.zeros((16,), jnp.float32)
        return 0
    lax.fori_loop(0, nrows, row, 0)


def _deg_body(ei_hbm, out_hbm, idx_v, ones_v, zero_v, acc_sh, dsem):
    c = lax.axis_index("c")
    s = lax.axis_index("s")
    wid = c * NS + s

    def ones_row(i, _):
        for k in range(DEG_W // 16):
            ones_v[i, pl.ds(k * 16, 16)] = jnp.full((16,), 1.0, jnp.float32)
        return 0
    lax.fori_loop(0, CHUNK, ones_row, 0)
    _zero_fill(zero_v, 64, DEG_W)

    def zero_acc(k, _):
        pltpu.sync_copy(zero_v, acc_sh.at[pl.ds(s * RPT + k * 64, 64)])
        return 0
    lax.fori_loop(0, RPT // 64, zero_acc, 0)
    plsc.subcore_barrier()

    pltpu.sync_copy(ei_hbm.at[1, pl.ds(wid * CPW, CPW), :], idx_v)

    def body(j, _):
        pltpu.async_copy(ones_v, acc_sh.at[idx_v.at[j]], dsem, add=True)
        return 0
    lax.fori_loop(0, CPW, body, 0)

    def drain(j, _):
        pltpu.make_async_copy(ones_v, acc_sh.at[idx_v.at[0]], dsem).wait()
        return 0
    lax.fori_loop(0, CPW, drain, 0)
    plsc.subcore_barrier()

    pltpu.sync_copy(acc_sh.at[pl.ds(s * RPT, RPT)],
                    out_hbm.at[c, pl.ds(s * RPT, RPT)])


def _agg_body(hs_hbm, src_hbm, dst_hbm, out_hbm,
              src_v, dst_v, rows0, rows1, acc_sh,
              gsem0, gsem1, ssem0, ssem1):
    c = lax.axis_index("c")
    s = lax.axis_index("s")
    wid = c * NS + s

    _zero_fill(rows0, CHUNK, H)

    def zero_acc(k, _):
        pltpu.sync_copy(rows0, acc_sh.at[pl.ds(s * RPT + k * CHUNK, CHUNK)])
        return 0
    lax.fori_loop(0, RPT // CHUNK, zero_acc, 0)
    plsc.subcore_barrier()

    rows = (rows0, rows1)
    gsems = (gsem0, gsem1)
    ssems = (ssem0, ssem1)

    def sidx(r):
        return src_v.at[r]

    def didx(r):
        return dst_v.at[r]

    for q in range(CPW // IB):
        pltpu.sync_copy(src_hbm.at[pl.ds(wid * CPW + q * IB, IB)], src_v)
        pltpu.sync_copy(dst_hbm.at[pl.ds(wid * CPW + q * IB, IB)], dst_v)
        for b in range(2):
            pltpu.async_copy(hs_hbm.at[sidx(b)], rows[b], gsems[b])

        def pair(p, _):
            for b in range(2):
                r = 2 * p + b
                pltpu.make_async_copy(hs_hbm.at[sidx(0)],
                                      rows[b], gsems[b]).wait()
                pltpu.async_copy(rows[b], acc_sh.at[didx(r)], ssems[b],
                                 add=True)
                nr = r + 2

                @pl.when(nr < IB)
                def _():
                    pltpu.make_async_copy(rows[b], acc_sh.at[didx(r)],
                                          ssems[b]).wait()
                    pltpu.async_copy(hs_hbm.at[sidx(nr)], rows[b], gsems[b])
            return 0
        lax.fori_loop(0, IB // 2, pair, 0)
        for b in range(2):
            pltpu.make_async_copy(rows[b], acc_sh.at[didx(0)],
                                  ssems[b]).wait()
    plsc.subcore_barrier()

    pltpu.sync_copy(acc_sh.at[pl.ds(s * RPT, RPT)],
                    out_hbm.at[c, pl.ds(s * RPT, RPT)])


@functools.cache
def _sc_kernels():
    mesh = plsc.VectorSubcoreMesh(core_axis_name="c", subcore_axis_name="s",
                                  num_cores=NC, num_subcores=NS)
    deg_k = pl.kernel(
        _deg_body,
        out_type=jax.ShapeDtypeStruct((NC, NP, DEG_W), jnp.float32),
        mesh=mesh,
        scratch_types=[
            pltpu.VMEM((CPW, CHUNK), jnp.int32),
            pltpu.VMEM((CHUNK, DEG_W), jnp.float32),
            pltpu.VMEM((64, DEG_W), jnp.float32),
            pltpu.VMEM_SHARED((NP, DEG_W), jnp.float32),
            pltpu.SemaphoreType.DMA,
        ],
    )
    agg_k = pl.kernel(
        _agg_body,
        out_type=jax.ShapeDtypeStruct((NC, NP, H), jnp.float32),
        mesh=mesh,
        scratch_types=[
            pltpu.VMEM((IB, CHUNK), jnp.int32),
            pltpu.VMEM((IB, CHUNK), jnp.int32),
            pltpu.VMEM((CHUNK, H), jnp.float32),
            pltpu.VMEM((CHUNK, H), jnp.float32),
            pltpu.VMEM_SHARED((NP, H), jnp.float32),
            pltpu.SemaphoreType.DMA,
            pltpu.SemaphoreType.DMA,
            pltpu.SemaphoreType.DMA,
            pltpu.SemaphoreType.DMA,
        ],
    )
    return deg_k, agg_k


BLK = 2048


def _k1_body(degtab_ref, x_ref, w1_ref, hs_ref, dinv_ref):
    deg = degtab_ref[0, :, 0:1] + degtab_ref[1, :, 0:1]
    dinv = lax.rsqrt(deg + 1.0)
    h = jnp.dot(x_ref[...], w1_ref[...], preferred_element_type=jnp.float32)
    hs_ref[...] = h * dinv
    dinv_ref[...] = dinv


def _k2_body(acc_ref, hs_ref, dinv_ref, b_ref, w_ref, out_ref):
    dinv = dinv_ref[...]
    z = jnp.maximum((acc_ref[0] + acc_ref[1] + hs_ref[...]) * dinv + b_ref[...],
                    0.0)
    h2 = jnp.dot(z, w_ref[...], preferred_element_type=jnp.float32)
    out_ref[...] = h2 * dinv


def _k3_body(acc_ref, hs_ref, dinv_ref, b_ref, w3_ref, b3_ref, out_ref):
    dinv = dinv_ref[...]
    z = jnp.maximum((acc_ref[0] + acc_ref[1] + hs_ref[...]) * dinv + b_ref[...],
                    0.0)
    logits = jnp.dot(z, w3_ref[...], preferred_element_type=jnp.float32)
    logits = logits + b3_ref[...]
    m = jnp.max(logits, axis=1, keepdims=True)
    e = jnp.exp(logits - m)
    sm = e / jnp.sum(e, axis=1, keepdims=True)
    out_ref[...] = sm[:, :8]


_row_spec = pl.BlockSpec((BLK, 128), lambda i: (i, 0))
_vec_spec = pl.BlockSpec((BLK, 1), lambda i: (i, 0))
_mat_spec = pl.BlockSpec((128, 128), lambda i: (0, 0))
_bias_spec = pl.BlockSpec((1, 128), lambda i: (0, 0))
_acc_spec = pl.BlockSpec((NC, BLK, 128), lambda i: (0, i, 0))

_k1 = pl.pallas_call(
    _k1_body,
    grid=(NP // BLK,),
    in_specs=[_acc_spec, _row_spec, _mat_spec],
    out_specs=[_row_spec, _vec_spec],
    out_shape=[jax.ShapeDtypeStruct((NP, H), jnp.float32),
               jax.ShapeDtypeStruct((NP, 1), jnp.float32)],
)

_k2 = pl.pallas_call(
    _k2_body,
    grid=(NP // BLK,),
    in_specs=[_acc_spec, _row_spec, _vec_spec, _bias_spec, _mat_spec],
    out_specs=_row_spec,
    out_shape=jax.ShapeDtypeStruct((NP, H), jnp.float32),
)

_k3 = pl.pallas_call(
    _k3_body,
    grid=(NP // BLK,),
    in_specs=[_acc_spec, _row_spec, _vec_spec, _bias_spec, _mat_spec,
              _bias_spec],
    out_specs=pl.BlockSpec((BLK, 8), lambda i: (i, 0)),
    out_shape=jax.ShapeDtypeStruct((NP, 8), jnp.float32),
)


def kernel(x, edge_index, W1, b1, W2, b2, W3, b3):
    ei3 = edge_index.reshape(2, NCH, CHUNK)
    pad_src = jnp.arange(PADC * CHUNK, dtype=jnp.int32) % N
    pad_dst = N + jnp.arange(PADC * CHUNK, dtype=jnp.int32) % (NP - N)
    pad3 = jnp.stack([pad_src.reshape(PADC, CHUNK),
                      pad_dst.reshape(PADC, CHUNK)])
    ei4 = jnp.concatenate([ei3, pad3], axis=1)
    x_p = jnp.pad(x, ((0, NP - N), (0, 0)))

    src4 = ei4[0]
    dst4 = ei4[1]

    _deg_kernel, _agg_kernel = _sc_kernels()
    degtab = _deg_kernel(ei4)

    hs1, dinv = _k1(degtab, x_p, W1)
    acc1 = _agg_kernel(hs1, src4, dst4)
    hs2 = _k2(acc1, hs1, dinv, b1.reshape(1, H), W2)
    acc2 = _agg_kernel(hs2, src4, dst4)

    w3p = jnp.pad(W3, ((0, 0), (0, 128 - C)))
    b3p = jnp.concatenate([b3, jnp.full((128 - C,), -1e30, jnp.float32)])
    out = _k3(acc2, hs2, dinv, b2.reshape(1, H), w3p, b3p.reshape(1, 128))
    return out[:N, :C]
